# Initial kernel scaffold; baseline (speedup 1.0000x reference)
import jax
import jax.numpy as jnp
from jax.experimental import pallas as pl


def kernel(input_ids, wte, wpe):
    raise NotImplementedError("write your pallas kernel here")


# speedup vs baseline: 1.1379x; 1.1379x over previous
import functools

import jax
import jax.numpy as jnp
from jax import lax
from jax.experimental import pallas as pl
from jax.experimental.pallas import tpu as pltpu
from jax.experimental.pallas import tpu_sc as plsc

_VOCAB = 50257
_N_POS = 2048
_D = 768
_B = 4
_S = 2048
_NW = 32
_SPW = _S // _NW
_LANES = 16


def _emb_body(ids_hbm, wte_hbm, wpe_hbm, out_hbm, idx_v, wpe_v, rows_v, sem):
    cid = lax.axis_index("c")
    sid = lax.axis_index("s")
    wid = sid * 2 + cid
    s_base = wid * _SPW

    pltpu.sync_copy(wpe_hbm.at[pl.ds(s_base, _SPW)], wpe_v)
    for b in range(_B):
        pltpu.sync_copy(ids_hbm.at[b, pl.ds(s_base, _SPW)], idx_v.at[b])

    for b in range(_B):
        pltpu.async_copy(wte_hbm.at[idx_v.at[b]], rows_v, sem).wait()

        def row_add(r, carry):
            for c in range(_D // _LANES):
                sl = pl.ds(c * _LANES, _LANES)
                rows_v[r, sl] = rows_v[r, sl] + wpe_v[r, sl]
            return carry

        lax.fori_loop(0, _SPW, row_add, 0)
        pltpu.sync_copy(rows_v, out_hbm.at[b, pl.ds(s_base, _SPW)])


_emb = functools.partial(
    pl.kernel,
    out_type=jax.ShapeDtypeStruct((_B, _S, _D), jnp.float32),
    mesh=plsc.VectorSubcoreMesh(core_axis_name="c", subcore_axis_name="s"),
    scratch_types=[
        pltpu.VMEM((_B, _SPW), jnp.int32),
        pltpu.VMEM((_SPW, _D), jnp.float32),
        pltpu.VMEM((_SPW, _D), jnp.float32),
        pltpu.SemaphoreType.DMA,
    ],
)(_emb_body)


def kernel(input_ids, wte, wpe):
    ids = jnp.asarray(input_ids, jnp.int32)
    return _emb(ids, wte, wpe)
